# baseline (speedup 1.0000x reference)
import functools

import jax
import jax.numpy as jnp
from jax.experimental import pallas as pl
from jax.experimental.pallas import tpu as pltpu


def _conv4_kernel(x_ref, w_ref, b_ref, o_ref, buf, *, H, W, C, block_count):
    P = W // 2
    R = H * P
    C2, C4, C6 = 2 * C, 4 * C, 6 * C
    BM = min(512, R)

    buf[:, pl.ds(0, P), :] = jnp.zeros((2, P, C2), jnp.bfloat16)
    buf[:, pl.ds(P + R, P), :] = jnp.zeros((2, P, C2), jnp.bfloat16)
    buf[0, pl.ds(P, R), :] = x_ref[0]

    w_all = w_ref[...]
    bias = b_ref[0, :].astype(jnp.float32)
    BH = BM // P

    for l in range(block_count):
        src = l % 2
        dst = 1 - src
        for m in range(0, R, BM):
            patch = jnp.concatenate(
                [buf[src, pl.ds(kh * P + m, BM), :] for kh in range(3)],
                axis=1)
            acc = jnp.dot(patch, w_all,
                          preferred_element_type=jnp.float32)
            a = acc.reshape(BH, P, C4)
            ao = a[:, :, 0:C]
            ae = a[:, :, C:C2]
            co = a[:, :, C2:3 * C]
            de = a[:, :, 3 * C:C4]
            zp = jnp.zeros((BH, 1, C), jnp.float32)
            ta = jnp.concatenate([ao[:, 1:, :], zp], axis=1)
            td = jnp.concatenate([zp, de[:, :-1, :]], axis=1)
            z = jnp.maximum(
                jnp.concatenate([ae + td, co + ta], axis=2) + bias, 0.0)
            if l < block_count - 1:
                buf[dst, pl.ds(P + m, BM), :] = (
                    z.reshape(BM, C2).astype(jnp.bfloat16))
            else:
                o_ref[0, pl.ds(m, BM), :] = z.reshape(BM, C2)


def kernel(x, w, b):
    N, C, H, W = x.shape
    block_count = 4
    x_flat = jnp.transpose(x, (0, 2, 3, 1)).reshape(N, H * W // 2, 2 * C)
    x_flat = x_flat.astype(jnp.bfloat16)
    zb = jnp.zeros((C, C), w.dtype)
    w_all = jnp.concatenate([
        jnp.concatenate(
            [jnp.concatenate([w[kh, 2], w[kh, 1], w[kh, 0], zb], axis=1),
             jnp.concatenate([zb, w[kh, 2], w[kh, 1], w[kh, 0]], axis=1)],
            axis=0)
        for kh in range(3)], axis=0)
    w_all = w_all.astype(jnp.bfloat16)
    b2 = jnp.concatenate([b, b]).reshape(1, 2 * C).astype(jnp.float32)

    kern = functools.partial(_conv4_kernel, H=H, W=W, C=C,
                             block_count=block_count)
    out_flat = pl.pallas_call(
        kern,
        out_shape=jax.ShapeDtypeStruct((N, H * W // 2, 2 * C), jnp.float32),
        grid=(N,),
        in_specs=[
            pl.BlockSpec((1, H * W // 2, 2 * C), lambda n: (n, 0, 0)),
            pl.BlockSpec((6 * C, 4 * C), lambda n: (0, 0)),
            pl.BlockSpec((1, 2 * C), lambda n: (0, 0)),
        ],
        out_specs=pl.BlockSpec((1, H * W // 2, 2 * C), lambda n: (n, 0, 0)),
        scratch_shapes=[
            pltpu.VMEM((2, (H + 2) * W // 2, 2 * C), jnp.bfloat16)],
        compiler_params=pltpu.CompilerParams(
            dimension_semantics=("parallel",)),
    )(x_flat, w_all, b2)

    return jnp.transpose(out_flat.reshape(N, H, W, C),
                         (0, 3, 1, 2)).astype(x.dtype)


# speedup vs baseline: 1.2961x; 1.2961x over previous
import functools

import jax
import jax.numpy as jnp
from jax.experimental import pallas as pl
from jax.experimental.pallas import tpu as pltpu

def _conv4_kernel(x_ref, w_ref, b_ref, o_ref, buf, *, H, W, C, block_count, G):
    HW = H * W
    PAD = W
    BM = min(1024, HW)

    buf[:, :, pl.ds(0, PAD), :] = jnp.zeros((G, 2, PAD, C), jnp.bfloat16)
    buf[:, :, pl.ds(PAD + HW, PAD), :] = jnp.zeros((G, 2, PAD, C),
                                                   jnp.bfloat16)
    for img in range(G):
        buf[img, 0, pl.ds(PAD, HW), :] = x_ref[img]

    w_all = w_ref[...]
    bias = b_ref[0, :].astype(jnp.float32)
    BH = BM // W

    for l in range(block_count):
        src = l % 2
        dst = 1 - src
        for m in range(0, HW, BM):
            for img in range(G):
                patch = jnp.concatenate(
                    [buf[img, src, pl.ds(kh * W + m, BM), :]
                     for kh in range(3)], axis=1)
                acc = jnp.dot(patch, w_all,
                              preferred_element_type=jnp.float32)
                a = acc.reshape(BH, W, 3 * C)
                a0 = a[:, :, 0:C]
                a1 = a[:, :, C:2 * C]
                a2 = a[:, :, 2 * C:3 * C]
                zcol = jnp.zeros((BH, 1, C), jnp.float32)
                t0 = jnp.concatenate([zcol, a0[:, :-1, :]], axis=1)
                t2 = jnp.concatenate([a2[:, 1:, :], zcol], axis=1)
                z = jnp.maximum(a1 + t0 + t2 + bias, 0.0)
                if l < block_count - 1:
                    buf[img, dst, pl.ds(PAD + m, BM), :] = (
                        z.reshape(BM, C).astype(jnp.bfloat16))
                else:
                    o_ref[img, pl.ds(m, BM), :] = z.reshape(BM, C)


def kernel(x, w, b):
    N, C, H, W = x.shape
    block_count = 4
    x_flat = jnp.transpose(x.astype(jnp.bfloat16),
                           (0, 2, 3, 1)).reshape(N, H * W, C)
    w_all = jnp.transpose(w, (0, 2, 1, 3)).reshape(3 * C, 3 * C)
    w_all = w_all.astype(jnp.bfloat16)
    b2 = b.reshape(1, C).astype(jnp.float32)

    g = 4 if N % 4 == 0 else 1
    kern = functools.partial(_conv4_kernel, H=H, W=W, C=C,
                             block_count=block_count, G=g)
    out_flat = pl.pallas_call(
        kern,
        out_shape=jax.ShapeDtypeStruct((N, H * W, C), jnp.float32),
        grid=(N // g,),
        in_specs=[
            pl.BlockSpec((g, H * W, C), lambda n: (n, 0, 0)),
            pl.BlockSpec((3 * C, 3 * C), lambda n: (0, 0)),
            pl.BlockSpec((1, C), lambda n: (0, 0)),
        ],
        out_specs=pl.BlockSpec((g, H * W, C), lambda n: (n, 0, 0)),
        scratch_shapes=[
            pltpu.VMEM((g, 2, (H + 2) * W, C), jnp.bfloat16)],
        compiler_params=pltpu.CompilerParams(
            dimension_semantics=("parallel",)),
    )(x_flat, w_all, b2)

    return jnp.transpose(out_flat.reshape(N, H, W, C),
                         (0, 3, 1, 2)).astype(x.dtype)


# speedup vs baseline: 1.3057x; 1.0074x over previous
import functools

import jax
import jax.numpy as jnp
from jax.experimental import pallas as pl
from jax.experimental.pallas import tpu as pltpu

def _conv4_kernel(x_ref, w_ref, b_ref, o_ref, buf, *, H, W, C, block_count, G):
    HW = H * W
    PAD = W
    BM = min(1024, HW)

    buf[:, :, pl.ds(0, PAD), :] = jnp.zeros((G, 2, PAD, C), jnp.bfloat16)
    buf[:, :, pl.ds(PAD + HW, PAD), :] = jnp.zeros((G, 2, PAD, C),
                                                   jnp.bfloat16)
    for img in range(G):
        buf[img, 0, pl.ds(PAD, HW), :] = x_ref[img]

    w_all = w_ref[...]
    bias = b_ref[0, :].astype(jnp.float32)
    BH = BM // W

    for l in range(block_count):
        src = l % 2
        dst = 1 - src
        for m in range(0, HW, BM):
            for img in range(G):
                patch = jnp.concatenate(
                    [buf[img, src, pl.ds(kh * W + m, BM), :]
                     for kh in range(3)], axis=1)
                acc = jnp.dot(patch, w_all,
                              preferred_element_type=jnp.float32)
                a = acc.reshape(BH, W, 3 * C)
                a0 = a[:, :, 0:C]
                a1 = a[:, :, C:2 * C]
                a2 = a[:, :, 2 * C:3 * C]
                zcol = jnp.zeros((BH, 1, C), jnp.float32)
                t0 = jnp.concatenate([zcol, a0[:, :-1, :]], axis=1)
                t2 = jnp.concatenate([a2[:, 1:, :], zcol], axis=1)
                z = jnp.maximum(a1 + t0 + t2 + bias, 0.0)
                if l < block_count - 1:
                    buf[img, dst, pl.ds(PAD + m, BM), :] = (
                        z.reshape(BM, C).astype(jnp.bfloat16))
                else:
                    o_ref[img, pl.ds(m, BM), :] = z.reshape(BM, C)


def kernel(x, w, b):
    N, C, H, W = x.shape
    block_count = 4
    x_flat = jnp.transpose(x.astype(jnp.bfloat16),
                           (0, 2, 3, 1)).reshape(N, H * W, C)
    w_all = jnp.transpose(w, (0, 2, 1, 3)).reshape(3 * C, 3 * C)
    w_all = w_all.astype(jnp.bfloat16)
    b2 = b.reshape(1, C).astype(jnp.float32)

    g = 2 if N % 2 == 0 else 1
    kern = functools.partial(_conv4_kernel, H=H, W=W, C=C,
                             block_count=block_count, G=g)
    out_flat = pl.pallas_call(
        kern,
        out_shape=jax.ShapeDtypeStruct((N, H * W, C), jnp.float32),
        grid=(N // g,),
        in_specs=[
            pl.BlockSpec((g, H * W, C), lambda n: (n, 0, 0)),
            pl.BlockSpec((3 * C, 3 * C), lambda n: (0, 0)),
            pl.BlockSpec((1, C), lambda n: (0, 0)),
        ],
        out_specs=pl.BlockSpec((g, H * W, C), lambda n: (n, 0, 0)),
        scratch_shapes=[
            pltpu.VMEM((g, 2, (H + 2) * W, C), jnp.bfloat16)],
        compiler_params=pltpu.CompilerParams(
            dimension_semantics=("parallel",)),
    )(x_flat, w_all, b2)

    return jnp.transpose(out_flat.reshape(N, H, W, C),
                         (0, 3, 1, 2)).astype(x.dtype)
